# baseline (speedup 1.0000x reference)
import jax
import jax.numpy as jnp
from jax.experimental import pallas as pl
from jax.experimental.pallas import tpu as pltpu

_CHUNK = 256


def _rule_kernel(mf_hbm, idxt_ref, firing_ref, norm_ref,
                 buf0, buf1, sem0, sem1):
    f, r = idxt_ref.shape
    b = firing_ref.shape[0]
    k = buf0.shape[1]
    m = k // f
    nch = b // _CHUNK
    mf2 = mf_hbm
    bufs = (buf0, buf1)
    sems = (sem0, sem1)

    def start(c):
        pltpu.make_async_copy(
            mf2.at[pl.ds(c * _CHUNK, _CHUNK), :], bufs[c % 2], sems[c % 2]
        ).start()

    start(0)

    idxt = idxt_ref[...]
    idx_exp = jnp.broadcast_to(idxt[:, None, :], (f, m, r)).reshape(k, r)
    m_of_k = jax.lax.broadcasted_iota(jnp.int32, (k, 1), 0) % m
    w = (idx_exp == m_of_k).astype(jnp.float32)

    for c in range(nch):
        pltpu.make_async_copy(
            mf2.at[pl.ds(c * _CHUNK, _CHUNK), :], bufs[c % 2], sems[c % 2]
        ).wait()
        if c + 1 < nch:
            start(c + 1)
        logs = jnp.log(bufs[c % 2][...] + 1e-9)
        log_firing = jax.lax.dot_general(
            logs, w, (((1,), (0,)), ((), ())),
            preferred_element_type=jnp.float32,
            precision=jax.lax.Precision.HIGHEST)
        firing = jnp.exp(log_firing)
        s = jnp.sum(firing, axis=1, keepdims=True) + 1e-6
        rows = pl.ds(c * _CHUNK, _CHUNK)
        firing_ref[rows, :] = firing
        norm_ref[rows, :] = firing / s


def kernel(mf_values, rule_indices):
    b, f, m = mf_values.shape
    r = rule_indices.shape[0]
    k = f * m
    idxt = rule_indices.astype(jnp.int32).T
    firing, norm = pl.pallas_call(
        _rule_kernel,
        in_specs=[
            pl.BlockSpec(memory_space=pl.ANY),
            pl.BlockSpec((f, r), lambda: (0, 0)),
        ],
        out_specs=(
            pl.BlockSpec((b, r), lambda: (0, 0)),
            pl.BlockSpec((b, r), lambda: (0, 0)),
        ),
        scratch_shapes=[
            pltpu.VMEM((_CHUNK, k), jnp.float32),
            pltpu.VMEM((_CHUNK, k), jnp.float32),
            pltpu.SemaphoreType.DMA,
            pltpu.SemaphoreType.DMA,
        ],
        out_shape=(jax.ShapeDtypeStruct((b, r), jnp.float32),
                   jax.ShapeDtypeStruct((b, r), jnp.float32)),
    )(jnp.reshape(mf_values, (b, k)), idxt)
    return firing, norm


# speedup vs baseline: 1.0011x; 1.0011x over previous
import jax
import jax.numpy as jnp
from jax.experimental import pallas as pl
from jax.experimental.pallas import tpu as pltpu

_CHUNK = 256
_NCH = 4


def _rule_kernel(mf_hbm, idxt_ref, firing_ref, norm_ref, *scratch):
    bufs = scratch[:_NCH]
    sems = scratch[_NCH:]
    f, r = idxt_ref.shape
    k = bufs[0].shape[1]
    m = k // f

    copies = [
        pltpu.make_async_copy(
            mf_hbm.at[pl.ds(c * _CHUNK, _CHUNK), :], bufs[c], sems[c])
        for c in range(_NCH)
    ]
    for cp in copies:
        cp.start()

    idxt = idxt_ref[...]
    idx_exp = jnp.broadcast_to(idxt[:, None, :], (f, m, r)).reshape(k, r)
    m_of_k = jax.lax.broadcasted_iota(jnp.int32, (k, 1), 0) % m
    w = (idx_exp == m_of_k).astype(jnp.float32)

    for c in range(_NCH):
        copies[c].wait()
        logs = jnp.log(bufs[c][...] + 1e-9)
        log_firing = jax.lax.dot_general(
            logs, w, (((1,), (0,)), ((), ())),
            preferred_element_type=jnp.float32,
            precision=jax.lax.Precision.HIGHEST)
        firing = jnp.exp(log_firing)
        s = jnp.sum(firing, axis=1, keepdims=True) + 1e-6
        rows = pl.ds(c * _CHUNK, _CHUNK)
        firing_ref[rows, :] = firing
        norm_ref[rows, :] = firing / s


def kernel(mf_values, rule_indices):
    b, f, m = mf_values.shape
    r = rule_indices.shape[0]
    k = f * m
    idxt = rule_indices.astype(jnp.int32).T
    firing, norm = pl.pallas_call(
        _rule_kernel,
        in_specs=[
            pl.BlockSpec(memory_space=pl.ANY),
            pl.BlockSpec((f, r), lambda: (0, 0)),
        ],
        out_specs=(
            pl.BlockSpec((b, r), lambda: (0, 0)),
            pl.BlockSpec((b, r), lambda: (0, 0)),
        ),
        scratch_shapes=(
            [pltpu.VMEM((_CHUNK, k), jnp.float32) for _ in range(_NCH)]
            + [pltpu.SemaphoreType.DMA for _ in range(_NCH)]
        ),
        out_shape=(jax.ShapeDtypeStruct((b, r), jnp.float32),
                   jax.ShapeDtypeStruct((b, r), jnp.float32)),
    )(jnp.reshape(mf_values, (b, k)), idxt)
    return firing, norm


# speedup vs baseline: 1.2736x; 1.2721x over previous
import jax
import jax.numpy as jnp
from jax.experimental import pallas as pl
from jax.experimental.pallas import tpu as pltpu


def _rule_kernel(mf_ref, idxt_ref, firing_ref, norm_ref):
    f, r = idxt_ref.shape
    k = mf_ref.shape[1]
    m = k // f
    idxt = idxt_ref[...]
    idx_exp = jnp.broadcast_to(idxt[:, None, :], (f, m, r)).reshape(k, r)
    m_of_k = jax.lax.broadcasted_iota(jnp.int32, (k, 1), 0) % m
    w = (idx_exp == m_of_k).astype(jnp.float32)
    logs = jnp.log(mf_ref[...] + 1e-9)
    log_firing = jax.lax.dot_general(
        logs, w, (((1,), (0,)), ((), ())),
        preferred_element_type=jnp.float32,
        precision=jax.lax.Precision.HIGHEST)
    firing = jnp.exp(log_firing)
    s = jnp.sum(firing, axis=1, keepdims=True) + 1e-6
    firing_ref[...] = firing
    norm_ref[...] = firing / s


def kernel(mf_values, rule_indices):
    b, f, m = mf_values.shape
    r = rule_indices.shape[0]
    mf_flat = jnp.reshape(mf_values, (b, f * m))
    idxt = rule_indices.astype(jnp.int32).T
    firing, norm = pl.pallas_call(
        _rule_kernel,
        compiler_params=pltpu.CompilerParams(
            allow_input_fusion=[True, False]),
        out_shape=(jax.ShapeDtypeStruct((b, r), jnp.float32),
                   jax.ShapeDtypeStruct((b, r), jnp.float32)),
    )(mf_flat, idxt)
    return firing, norm
